# baseline (speedup 1.0000x reference)
import functools

import jax
import jax.numpy as jnp
from jax import lax
from jax.experimental import pallas as pl
from jax.experimental.pallas import tpu as pltpu
from jax.experimental.pallas import tpu_sc as plsc

_B, _N, _D = 1024, 1000, 128


def _sc_gather(current_node, capacity, used_capacity, psi_flat, coords_zc):
    info = plsc.get_sparse_core_info()
    nc, ns, nl = info.num_cores, info.num_subcores, info.num_lanes
    nw = nc * ns
    bpw = _B // nw

    mesh = plsc.VectorSubcoreMesh(core_axis_name="c", subcore_axis_name="s")

    @functools.partial(
        pl.kernel,
        out_type=(
            jax.ShapeDtypeStruct((_B, _D), jnp.float32),
            jax.ShapeDtypeStruct((_B, 4), jnp.float32),
        ),
        mesh=mesh,
        scratch_types=[
            pltpu.VMEM((bpw,), jnp.int32),
            pltpu.VMEM((2 * bpw,), jnp.int32),
            pltpu.VMEM((bpw,), jnp.float32),
            pltpu.VMEM((bpw,), jnp.float32),
            pltpu.VMEM((bpw, _D), jnp.float32),
            pltpu.VMEM((2 * bpw, 128), jnp.float32),
            pltpu.VMEM((bpw, 4), jnp.float32),
            pltpu.SemaphoreType.DMA,
            pltpu.SemaphoreType.DMA,
        ],
        compiler_params=pltpu.CompilerParams(needs_layout_passes=False),
    )
    def gather_kernel(node_hbm, cap_hbm, used_hbm, psi_hbm, zc_hbm,
                      psi_out, ex_out,
                      idx_v, idx2_v, cap_v, used_v, rows_v, cbuf_v, ex_v,
                      sem_a, sem_b):
        wid = lax.axis_index("s") * nc + lax.axis_index("c")
        base = wid * bpw
        btile2 = lax.shift_right_logical(base, 7) * 2
        lane0 = base & 127
        pltpu.sync_copy(node_hbm.at[pl.ds(base, bpw)], idx_v)
        pltpu.sync_copy(cap_hbm.at[pl.ds(base, bpw)], cap_v)
        pltpu.sync_copy(used_hbm.at[pl.ds(base, bpw)], used_v)
        for j in range(bpw // nl):
            sl = pl.ds(j * nl, nl)
            iv = j * nl + lax.iota(jnp.int32, nl)
            node = idx_v[sl]
            crow = node * 16 + btile2
            idx2_v[sl] = crow
            idx2_v[pl.ds(bpw + j * nl, nl)] = crow + 1
            idx_v[sl] = (base + iv) * _N + node
            cap = cap_v[sl]
            cap_norm = (cap - used_v[sl]) / jnp.maximum(cap, 1e-8)
            live = jnp.where(node != 0, 1.0, 0.0).astype(jnp.float32)
            plsc.store_scatter(ex_v, [iv, jnp.full((nl,), 0, jnp.int32)],
                               cap_norm)
            plsc.store_scatter(ex_v, [iv, jnp.full((nl,), 1, jnp.int32)],
                               live)
        cp_a = pltpu.async_copy(psi_hbm.at[idx_v], rows_v, sem_a)
        cp_b = pltpu.async_copy(zc_hbm.at[idx2_v], cbuf_v, sem_b)
        cp_b.wait()
        for c in range(2):
            for h in range(bpw // nl):
                iv = h * nl + lax.iota(jnp.int32, nl)
                vals = plsc.load_gather(cbuf_v, [c * bpw + iv, lane0 + iv])
                plsc.store_scatter(
                    ex_v, [iv, jnp.full((nl,), 2 + c, jnp.int32)], vals)
        pltpu.sync_copy(ex_v, ex_out.at[pl.ds(base, bpw)])
        cp_a.wait()
        pltpu.sync_copy(rows_v, psi_out.at[pl.ds(base, bpw)])

    return gather_kernel(current_node, capacity, used_capacity, psi_flat,
                         coords_zc)


def _tc_project_body(psi_ref, ex_ref, tf_ref, wqt_ref, q_ref):
    ex = ex_ref[...]
    psi = psi_ref[...] * ex[:, 1:2]
    q = lax.dot_general(psi, wqt_ref[0:_D, :],
                        (((1,), (0,)), ((), ())),
                        preferred_element_type=jnp.float32)
    t_col = jnp.full((_B, 1), tf_ref[0, 0], jnp.float32)
    extras = jnp.concatenate([ex[:, 0:1], t_col, ex[:, 2:4]], axis=1)
    q = q + lax.dot_general(extras, wqt_ref[_D:, :],
                            (((1,), (0,)), ((), ())),
                            preferred_element_type=jnp.float32)
    q_ref[...] = q


def _tc_project(psi_curr, ex, t_frac, wqt):
    return pl.pallas_call(
        _tc_project_body,
        out_shape=jax.ShapeDtypeStruct((_B, _D), jnp.float32),
        in_specs=[
            pl.BlockSpec(memory_space=pltpu.VMEM),
            pl.BlockSpec(memory_space=pltpu.VMEM),
            pl.BlockSpec(memory_space=pltpu.SMEM),
            pl.BlockSpec(memory_space=pltpu.VMEM),
        ],
        out_specs=pl.BlockSpec(memory_space=pltpu.VMEM),
    )(psi_curr, ex, t_frac, wqt)


def kernel(psi_prime, current_node, capacity, used_capacity, coords, step,
           n_customers, Wq):
    psi_flat = psi_prime.reshape(_B * _N, _D)
    zc = (coords.transpose(1, 0, 2).reshape(_N, 8, 128, 2)
          .transpose(0, 1, 3, 2).reshape(_N * 16, 128))
    psi_curr, ex = _sc_gather(current_node, capacity, used_capacity,
                              psi_flat, zc)

    t_frac = (jnp.asarray(step, jnp.float32)
              / jnp.maximum(jnp.asarray(n_customers, jnp.float32), 1.0))
    t_frac = t_frac.reshape(1, 1)
    query = _tc_project(psi_curr, ex, t_frac, Wq.T)
    current_coords = ex[:, 2:4]
    return (query, current_coords)


# speedup vs baseline: 1.1017x; 1.1017x over previous
import functools

import jax
import jax.numpy as jnp
from jax import lax
from jax.experimental import pallas as pl
from jax.experimental.pallas import tpu as pltpu
from jax.experimental.pallas import tpu_sc as plsc

_B, _N, _D = 1024, 1000, 128


def _sc_gather(current_node, capacity, used_capacity, psi_flat, coords_zc):
    info = plsc.get_sparse_core_info()
    nc, ns, nl = info.num_cores, info.num_subcores, info.num_lanes
    nw = nc * ns
    bpw = _B // nw

    mesh = plsc.VectorSubcoreMesh(core_axis_name="c", subcore_axis_name="s")

    @functools.partial(
        pl.kernel,
        out_type=(
            jax.ShapeDtypeStruct((_B, _D), jnp.float32),
            jax.ShapeDtypeStruct((_B, 4), jnp.float32),
        ),
        mesh=mesh,
        scratch_types=[
            pltpu.VMEM((bpw,), jnp.int32),
            pltpu.VMEM((2 * bpw,), jnp.int32),
            pltpu.VMEM((bpw,), jnp.float32),
            pltpu.VMEM((bpw,), jnp.float32),
            pltpu.VMEM((bpw, _D), jnp.float32),
            pltpu.VMEM((2 * bpw, 128), jnp.float32),
            pltpu.VMEM((bpw, 4), jnp.float32),
            pltpu.SemaphoreType.DMA,
            pltpu.SemaphoreType.DMA,
        ],
        compiler_params=pltpu.CompilerParams(needs_layout_passes=False),
    )
    def gather_kernel(node_hbm, cap_hbm, used_hbm, psi_hbm, zc_hbm,
                      psi_out, ex_out,
                      idx_v, idx2_v, cap_v, used_v, rows_v, cbuf_v, ex_v,
                      sem_a, sem_b):
        wid = lax.axis_index("s") * nc + lax.axis_index("c")
        base = wid * bpw
        btile2 = lax.shift_right_logical(base, 7) * 2
        lane0 = base & 127
        cp_n = pltpu.async_copy(node_hbm.at[pl.ds(base, bpw)], idx_v, sem_a)
        cp_c = pltpu.async_copy(cap_hbm.at[pl.ds(base, bpw)], cap_v, sem_a)
        cp_u = pltpu.async_copy(used_hbm.at[pl.ds(base, bpw)], used_v, sem_a)
        cp_n.wait()
        cp_c.wait()
        cp_u.wait()
        for j in range(bpw // nl):
            sl = pl.ds(j * nl, nl)
            iv = j * nl + lax.iota(jnp.int32, nl)
            node = idx_v[sl]
            crow = node * 16 + btile2
            idx2_v[sl] = crow
            idx2_v[pl.ds(bpw + j * nl, nl)] = crow + 1
            idx_v[sl] = (base + iv) * _N + node
            cap = cap_v[sl]
            cap_norm = (cap - used_v[sl]) / jnp.maximum(cap, 1e-8)
            live = jnp.where(node != 0, 1.0, 0.0).astype(jnp.float32)
            plsc.store_scatter(ex_v, [iv, jnp.full((nl,), 0, jnp.int32)],
                               cap_norm)
            plsc.store_scatter(ex_v, [iv, jnp.full((nl,), 1, jnp.int32)],
                               live)
        cp_a = pltpu.async_copy(psi_hbm.at[idx_v], rows_v, sem_a)
        cp_b = pltpu.async_copy(zc_hbm.at[idx2_v], cbuf_v, sem_b)
        cp_b.wait()
        for c in range(2):
            for h in range(bpw // nl):
                iv = h * nl + lax.iota(jnp.int32, nl)
                vals = plsc.load_gather(cbuf_v, [c * bpw + iv, lane0 + iv])
                plsc.store_scatter(
                    ex_v, [iv, jnp.full((nl,), 2 + c, jnp.int32)], vals)
        pltpu.sync_copy(ex_v, ex_out.at[pl.ds(base, bpw)])
        cp_a.wait()
        pltpu.sync_copy(rows_v, psi_out.at[pl.ds(base, bpw)])

    return gather_kernel(current_node, capacity, used_capacity, psi_flat,
                         coords_zc)


def _tc_project_body(psi_ref, ex_ref, tf_ref, wqt_ref, q_ref, cc2_ref):
    ex = ex_ref[...]
    psi = psi_ref[...] * ex[:, 1:2]
    q = lax.dot_general(psi, wqt_ref[0:_D, :],
                        (((1,), (0,)), ((), ())),
                        preferred_element_type=jnp.float32)
    t_col = jnp.full((_B, 1), tf_ref[0, 0], jnp.float32)
    extras = jnp.concatenate([ex[:, 0:1], t_col, ex[:, 2:4]], axis=1)
    q = q + lax.dot_general(extras, wqt_ref[_D:, :],
                            (((1,), (0,)), ((), ())),
                            preferred_element_type=jnp.float32)
    q_ref[...] = q
    cc = ex[:, 2:4].reshape(8, 128, 2)
    cc2_ref[...] = jnp.transpose(cc, (0, 2, 1)).reshape(16, 128)


def _tc_project(psi_curr, ex, t_frac, wqt):
    return pl.pallas_call(
        _tc_project_body,
        out_shape=(
            jax.ShapeDtypeStruct((_B, _D), jnp.float32),
            jax.ShapeDtypeStruct((16, 128), jnp.float32),
        ),
        in_specs=[
            pl.BlockSpec(memory_space=pltpu.VMEM),
            pl.BlockSpec(memory_space=pltpu.VMEM),
            pl.BlockSpec(memory_space=pltpu.SMEM),
            pl.BlockSpec(memory_space=pltpu.VMEM),
        ],
        out_specs=(
            pl.BlockSpec(memory_space=pltpu.VMEM),
            pl.BlockSpec(memory_space=pltpu.VMEM),
        ),
    )(psi_curr, ex, t_frac, wqt)


def kernel(psi_prime, current_node, capacity, used_capacity, coords, step,
           n_customers, Wq):
    psi_flat = psi_prime.reshape(_B * _N, _D)
    zc = (coords.transpose(1, 0, 2).reshape(_N, 8, 128, 2)
          .transpose(0, 1, 3, 2).reshape(_N * 16, 128))
    psi_curr, ex = _sc_gather(current_node, capacity, used_capacity,
                              psi_flat, zc)

    t_frac = (jnp.asarray(step, jnp.float32)
              / jnp.maximum(jnp.asarray(n_customers, jnp.float32), 1.0))
    t_frac = t_frac.reshape(1, 1)
    query, cc16 = _tc_project(psi_curr, ex, t_frac, Wq.T)
    current_coords = (cc16.reshape(8, 2, 128).transpose(0, 2, 1)
                      .reshape(_B, 2))
    return (query, current_coords)
